# baseline (speedup 1.0000x reference)
import functools

import jax
import jax.numpy as jnp
from jax import lax
from jax.experimental import pallas as pl
from jax.experimental.pallas import tpu as pltpu
from jax.experimental.pallas import tpu_sc as plsc

BT = 128
D_IN = 768
D_SAE = 24576
TOPK = 64
F_BLK = 2048
NB = D_SAE // F_BLK

NC, NS, L = 2, 16, 16
NW = NC * NS
ROWS_W = BT // NW
NV = D_SAE // L
CD = D_IN // L


def _tc_body(x_ref, we_ref, be_ref, bd_ref, sparse_ref, acts_ref, t_ref, j_ref):
    step = pl.program_id(0)

    @pl.when(step < NB)
    def _encode():
        xc = x_ref[...] - bd_ref[...]
        pre = lax.dot_general(xc, we_ref[...], (((1,), (0,)), ((), ())),
                              preferred_element_type=jnp.float32)
        pre = pre + be_ref[...]
        off = pl.multiple_of(step * F_BLK, F_BLK)
        acts_ref[:, pl.ds(off, F_BLK)] = jnp.maximum(pre, 0.0)

    @pl.when(step == NB)
    def _select():
        def count_ge(c):
            def chunk(i, acc):
                off = pl.multiple_of(i * F_BLK, F_BLK)
                bits = lax.bitcast_convert_type(
                    acts_ref[:, pl.ds(off, F_BLK)], jnp.int32)
                return acc + jnp.sum((bits >= c).astype(jnp.int32),
                                     axis=1, keepdims=True)
            return lax.fori_loop(0, NB, chunk,
                                 jnp.zeros((BT, 1), jnp.int32))

        def bit_step(i, t):
            c_test = t | jnp.left_shift(jnp.int32(1), 30 - i)
            cnt = count_ge(c_test)
            return jnp.where(cnt >= TOPK, c_test, t)
        t = lax.fori_loop(0, 31, bit_step, jnp.zeros((BT, 1), jnp.int32))

        n_gt = count_ge(t + 1)
        n_need = TOPK - n_gt

        def count_eq_below(c):
            def chunk(i, acc):
                off = pl.multiple_of(i * F_BLK, F_BLK)
                bits = lax.bitcast_convert_type(
                    acts_ref[:, pl.ds(off, F_BLK)], jnp.int32)
                idx = lax.broadcasted_iota(jnp.int32, (BT, F_BLK), 1) + off
                hit = (bits == t) & (idx < c)
                return acc + jnp.sum(hit.astype(jnp.int32),
                                     axis=1, keepdims=True)
            return lax.fori_loop(0, NB, chunk,
                                 jnp.zeros((BT, 1), jnp.int32))

        def idx_step(i, jcur):
            c_test = jcur | jnp.left_shift(jnp.int32(1), 14 - i)
            cnt = count_eq_below(c_test)
            return jnp.where(cnt < n_need, c_test, jcur)
        j = lax.fori_loop(0, 15, idx_step, jnp.zeros((BT, 1), jnp.int32))

        t_ref[...] = t
        j_ref[...] = j

    @pl.when(step >= NB)
    def _mask_write():
        blk = step - NB
        off = pl.multiple_of(blk * F_BLK, F_BLK)
        a = acts_ref[:, pl.ds(off, F_BLK)]
        bits = lax.bitcast_convert_type(a, jnp.int32)
        idx = lax.broadcasted_iota(jnp.int32, (BT, F_BLK), 1) + off
        keep = (bits > t_ref[...]) | ((bits == t_ref[...]) & (idx <= j_ref[...]))
        sparse_ref[...] = jnp.where(keep, a, 0.0)


def _tc_encode_select(x2d, w_enc, b_enc2d, b_dec2d):
    return pl.pallas_call(
        _tc_body,
        grid=(2 * NB,),
        in_specs=[
            pl.BlockSpec((BT, D_IN), lambda i: (0, 0)),
            pl.BlockSpec((D_IN, F_BLK), lambda i: (0, jnp.minimum(i, NB - 1))),
            pl.BlockSpec((1, F_BLK), lambda i: (0, jnp.minimum(i, NB - 1))),
            pl.BlockSpec((1, D_IN), lambda i: (0, 0)),
        ],
        out_specs=pl.BlockSpec((BT, F_BLK), lambda i: (0, jnp.maximum(i - NB, 0))),
        out_shape=jax.ShapeDtypeStruct((BT, D_SAE), jnp.float32),
        scratch_shapes=[
            pltpu.VMEM((BT, D_SAE), jnp.float32),
            pltpu.VMEM((BT, 1), jnp.int32),
            pltpu.VMEM((BT, 1), jnp.int32),
        ],
    )(x2d, w_enc, b_enc2d, b_dec2d)


def _sc_decode_body(sparse_hbm, wdec_hbm, bdec_hbm, out_hbm,
                    row_v, cidx_v, cval_v, rows_v, bvec_v, orow_v, sem):
    wid = lax.axis_index("s") * NC + lax.axis_index("c")
    zero16f = jnp.zeros((L,), jnp.float32)
    zero16i = jnp.zeros((L,), jnp.int32)
    lane = lax.iota(jnp.int32, L)

    pltpu.sync_copy(bdec_hbm, bvec_v)

    for r4 in range(ROWS_W):
        row = wid * ROWS_W + r4
        pltpu.sync_copy(sparse_hbm.at[row], row_v)

        for q in range(TOPK // L):
            cval_v[pl.ds(q * L, L)] = zero16f
            cidx_v[pl.ds(q * L, L)] = zero16i

        def scan_step(i, base):
            v = row_v[pl.ds(i * L, L)]
            m = v != 0.0
            cs = plsc.cumsum(m.astype(jnp.int32))
            pos = base + cs - 1
            plsc.store_scatter(cval_v, [pos], v, mask=m)
            plsc.store_scatter(cidx_v, [pos], lane + i * L, mask=m)
            pc = lax.gather(
                cs, jnp.full((L, 1), L - 1, jnp.int32),
                lax.GatherDimensionNumbers(
                    offset_dims=(), collapsed_slice_dims=(0,),
                    start_index_map=(0,)),
                (1,), mode=lax.GatherScatterMode.PROMISE_IN_BOUNDS)
            return base + pc
        lax.fori_loop(0, NV, scan_step, zero16i)

        pltpu.async_copy(wdec_hbm.at[cidx_v], rows_v, sem).wait()

        def c_step(c, carry):
            acc = zero16f
            for jg in range(TOPK // L):
                vals = cval_v[pl.ds(jg * L, L)]
                for ln in range(L):
                    vb = lax.gather(
                        vals, jnp.full((L, 1), ln, jnp.int32),
                        lax.GatherDimensionNumbers(
                            offset_dims=(), collapsed_slice_dims=(0,),
                            start_index_map=(0,)),
                        (1,), mode=lax.GatherScatterMode.PROMISE_IN_BOUNDS)
                    r = rows_v[jg * L + ln, pl.ds(c * L, L)]
                    acc = acc + vb * r
            orow_v[pl.ds(c * L, L)] = acc + bvec_v[pl.ds(c * L, L)]
            return carry
        lax.fori_loop(0, CD, c_step, jnp.int32(0))

        pltpu.sync_copy(orow_v, out_hbm.at[row])


def _sc_decode(sparse, w_dec, b_dec):
    mesh = plsc.VectorSubcoreMesh(core_axis_name="c", subcore_axis_name="s")
    f = pl.kernel(
        _sc_decode_body,
        mesh=mesh,
        compiler_params=pltpu.CompilerParams(needs_layout_passes=False),
        out_type=jax.ShapeDtypeStruct((BT, D_IN), jnp.float32),
        scratch_types=[
            pltpu.VMEM((D_SAE,), jnp.float32),
            pltpu.VMEM((TOPK,), jnp.int32),
            pltpu.VMEM((TOPK,), jnp.float32),
            pltpu.VMEM((TOPK, D_IN), jnp.float32),
            pltpu.VMEM((D_IN,), jnp.float32),
            pltpu.VMEM((D_IN,), jnp.float32),
            pltpu.SemaphoreType.DMA,
        ],
    )
    return f(sparse, w_dec, b_dec)


@jax.jit
def _run(x2d, w_enc, b_enc2d, w_dec, b_dec2d):
    sparse = _tc_encode_select(x2d, w_enc, b_enc2d, b_dec2d)
    recon = _sc_decode(sparse, w_dec, b_dec2d.reshape(-1))
    return recon, sparse


def kernel(x, W_enc, b_enc, W_dec, b_dec):
    b, s, d_in = x.shape
    x2d = x.reshape(b * s, d_in)
    recon, sparse = _run(x2d, W_enc, b_enc.reshape(1, -1),
                         W_dec, b_dec.reshape(1, -1))
    return recon.reshape(b, s, d_in), sparse.reshape(b, s, -1)


# speedup vs baseline: 1.1159x; 1.1159x over previous
import functools

import jax
import jax.numpy as jnp
from jax import lax
from jax.experimental import pallas as pl
from jax.experimental.pallas import tpu as pltpu
from jax.experimental.pallas import tpu_sc as plsc

BT = 128
D_IN = 768
D_SAE = 24576
TOPK = 64
F_BLK = 2048
NB = D_SAE // F_BLK

NC, NS, L = 2, 16, 16
NW = NC * NS
ROWS_W = BT // NW
NV = D_SAE // L
CD = D_IN // L


def _tc_body(x_ref, we_ref, be_ref, bd_ref, sparse_ref, acts_ref, t_ref, j_ref):
    step = pl.program_id(0)

    @pl.when(step < NB)
    def _encode():
        xc = x_ref[...] - bd_ref[...]
        pre = lax.dot_general(xc, we_ref[...], (((1,), (0,)), ((), ())),
                              preferred_element_type=jnp.float32)
        pre = pre + be_ref[...]
        off = pl.multiple_of(step * F_BLK, F_BLK)
        acts_ref[:, pl.ds(off, F_BLK)] = jnp.maximum(pre, 0.0)

    @pl.when(step == NB)
    def _select():
        def count_ge(c):
            def chunk(i, acc):
                off = pl.multiple_of(i * F_BLK, F_BLK)
                bits = lax.bitcast_convert_type(
                    acts_ref[:, pl.ds(off, F_BLK)], jnp.int32)
                return acc + jnp.sum((bits >= c).astype(jnp.int32),
                                     axis=1, keepdims=True)
            return lax.fori_loop(0, NB, chunk,
                                 jnp.zeros((BT, 1), jnp.int32))

        def bit_step(i, t):
            c_test = t | jnp.left_shift(jnp.int32(1), 30 - i)
            cnt = count_ge(c_test)
            return jnp.where(cnt >= TOPK, c_test, t)
        t = lax.fori_loop(0, 31, bit_step, jnp.zeros((BT, 1), jnp.int32))

        n_gt = count_ge(t + 1)
        n_need = TOPK - n_gt

        def count_eq_below(c):
            def chunk(i, acc):
                off = pl.multiple_of(i * F_BLK, F_BLK)
                bits = lax.bitcast_convert_type(
                    acts_ref[:, pl.ds(off, F_BLK)], jnp.int32)
                idx = lax.broadcasted_iota(jnp.int32, (BT, F_BLK), 1) + off
                hit = (bits == t) & (idx < c)
                return acc + jnp.sum(hit.astype(jnp.int32),
                                     axis=1, keepdims=True)
            return lax.fori_loop(0, NB, chunk,
                                 jnp.zeros((BT, 1), jnp.int32))

        def idx_step(i, jcur):
            c_test = jcur | jnp.left_shift(jnp.int32(1), 14 - i)
            cnt = count_eq_below(c_test)
            return jnp.where(cnt < n_need, c_test, jcur)
        j = lax.fori_loop(0, 15, idx_step, jnp.zeros((BT, 1), jnp.int32))

        t_ref[...] = t
        j_ref[...] = j

    @pl.when(step >= NB)
    def _mask_write():
        blk = step - NB
        off = pl.multiple_of(blk * F_BLK, F_BLK)
        a = acts_ref[:, pl.ds(off, F_BLK)]
        bits = lax.bitcast_convert_type(a, jnp.int32)
        idx = lax.broadcasted_iota(jnp.int32, (BT, F_BLK), 1) + off
        keep = (bits > t_ref[...]) | ((bits == t_ref[...]) & (idx <= j_ref[...]))
        sparse_ref[...] = jnp.where(keep, a, 0.0)


def _tc_encode_select(x2d, w_enc, b_enc2d, b_dec2d):
    return pl.pallas_call(
        _tc_body,
        grid=(2 * NB,),
        in_specs=[
            pl.BlockSpec((BT, D_IN), lambda i: (0, 0)),
            pl.BlockSpec((D_IN, F_BLK), lambda i: (0, jnp.minimum(i, NB - 1))),
            pl.BlockSpec((1, F_BLK), lambda i: (0, jnp.minimum(i, NB - 1))),
            pl.BlockSpec((1, D_IN), lambda i: (0, 0)),
        ],
        out_specs=pl.BlockSpec((BT, F_BLK), lambda i: (0, jnp.maximum(i - NB, 0))),
        out_shape=jax.ShapeDtypeStruct((BT, D_SAE), jnp.float32),
        scratch_shapes=[
            pltpu.VMEM((BT, D_SAE), jnp.float32),
            pltpu.VMEM((BT, 1), jnp.int32),
            pltpu.VMEM((BT, 1), jnp.int32),
        ],
    )(x2d, w_enc, b_enc2d, b_dec2d)


def _bcast_lane(vec, ln):
    return lax.gather(
        vec, jnp.full((L, 1), ln, jnp.int32),
        lax.GatherDimensionNumbers(
            offset_dims=(), collapsed_slice_dims=(0,), start_index_map=(0,)),
        (1,), mode=lax.GatherScatterMode.PROMISE_IN_BOUNDS)


def _sc_decode_body(sparse_hbm, wdec_hbm, bdec_hbm, out_hbm,
                    row_a, row_b, cidx_v, cval_v, rows_v, bvec_v, orow_v,
                    sem_r, sem_o, sem_g0, sem_g1, sem_g2, sem_g3):
    wid = lax.axis_index("s") * NC + lax.axis_index("c")
    zero16f = jnp.zeros((L,), jnp.float32)
    zero16i = jnp.zeros((L,), jnp.int32)
    lane = lax.iota(jnp.int32, L)
    sem_g = [sem_g0, sem_g1, sem_g2, sem_g3]
    G = 4
    NG = TOPK // L

    pltpu.sync_copy(bdec_hbm, bvec_v)
    row0 = wid * ROWS_W
    cp_in = pltpu.async_copy(sparse_hbm.at[row0], row_a, sem_r)
    cp_out = None

    for r4 in range(ROWS_W):
        row = row0 + r4
        buf = row_a if r4 % 2 == 0 else row_b
        nxt = row_b if r4 % 2 == 0 else row_a
        cp_in.wait()
        if r4 + 1 < ROWS_W:
            cp_in = pltpu.async_copy(sparse_hbm.at[row + 1], nxt, sem_r)

        for q in range(NG):
            cval_v[pl.ds(q * L, L)] = zero16f
            cidx_v[pl.ds(q * L, L)] = zero16i

        def scan_grp(g, base):
            vs = [buf[pl.ds((g * G + k) * L, L)] for k in range(G)]
            s = vs[0]
            for k in range(1, G):
                s = s + vs[k]
            hit = jnp.any(s != 0.0)

            def do_hit(b):
                for k in range(G):
                    v = vs[k]
                    m = v != 0.0
                    cs = plsc.cumsum(m.astype(jnp.int32))
                    pos = b + cs - 1
                    plsc.store_scatter(cval_v, [pos], v, mask=m)
                    plsc.store_scatter(cidx_v, [pos], lane + (g * G + k) * L,
                                       mask=m)
                    b = b + plsc.all_reduce_population_count(m)
                return b
            return lax.cond(hit, do_hit, lambda b: b, base)
        lax.fori_loop(0, NV // G, scan_grp, zero16i)

        cps = [pltpu.async_copy(wdec_hbm.at[cidx_v.at[pl.ds(q * L, L)]],
                                rows_v.at[pl.ds(q * L, L)], sem_g[q])
               for q in range(NG)]
        if cp_out is not None:
            cp_out.wait()

        def init_c(c, carry):
            orow_v[pl.ds(c * L, L)] = bvec_v[pl.ds(c * L, L)]
            return carry
        lax.fori_loop(0, CD, init_c, jnp.int32(0))

        for q in range(NG):
            cps[q].wait()
            vals = cval_v[pl.ds(q * L, L)]

            def c_step(c, carry):
                acc0 = zero16f
                acc1 = zero16f
                for ln in range(L):
                    vb = _bcast_lane(vals, ln)
                    r = rows_v[q * L + ln, pl.ds(c * L, L)]
                    if ln % 2 == 0:
                        acc0 = acc0 + vb * r
                    else:
                        acc1 = acc1 + vb * r
                orow_v[pl.ds(c * L, L)] += acc0 + acc1
                return carry
            lax.fori_loop(0, CD, c_step, jnp.int32(0))

        cp_out = pltpu.async_copy(orow_v, out_hbm.at[row], sem_o)
    cp_out.wait()


def _sc_decode(sparse, w_dec, b_dec):
    mesh = plsc.VectorSubcoreMesh(core_axis_name="c", subcore_axis_name="s")
    f = pl.kernel(
        _sc_decode_body,
        mesh=mesh,
        compiler_params=pltpu.CompilerParams(needs_layout_passes=False),
        out_type=jax.ShapeDtypeStruct((BT, D_IN), jnp.float32),
        scratch_types=[
            pltpu.VMEM((D_SAE,), jnp.float32),
            pltpu.VMEM((D_SAE,), jnp.float32),
            pltpu.VMEM((TOPK,), jnp.int32),
            pltpu.VMEM((TOPK,), jnp.float32),
            pltpu.VMEM((TOPK, D_IN), jnp.float32),
            pltpu.VMEM((D_IN,), jnp.float32),
            pltpu.VMEM((D_IN,), jnp.float32),
            pltpu.SemaphoreType.DMA,
            pltpu.SemaphoreType.DMA,
            pltpu.SemaphoreType.DMA,
            pltpu.SemaphoreType.DMA,
            pltpu.SemaphoreType.DMA,
            pltpu.SemaphoreType.DMA,
        ],
    )
    return f(sparse, w_dec, b_dec)


@jax.jit
def _run(x2d, w_enc, b_enc2d, w_dec, b_dec2d):
    sparse = _tc_encode_select(x2d, w_enc, b_enc2d, b_dec2d)
    recon = _sc_decode(sparse, w_dec, b_dec2d.reshape(-1))
    return recon, sparse


def kernel(x, W_enc, b_enc, W_dec, b_dec):
    b, s, d_in = x.shape
    x2d = x.reshape(b * s, d_in)
    recon, sparse = _run(x2d, W_enc, b_enc.reshape(1, -1),
                         W_dec, b_dec.reshape(1, -1))
    return recon.reshape(b, s, d_in), sparse.reshape(b, s, -1)
